# baseline (speedup 1.0000x reference)
import jax
import jax.numpy as jnp
from jax import lax
from jax.experimental import pallas as pl
from jax.experimental.pallas import tpu as pltpu
from jax.experimental.pallas import tpu_sc as plsc

B = 16384
D = 16
NC = 2
NS = 16
NW = NC * NS
BPW = B // NW
G = BPW // D

V = 1_000_000
VP = 1_000_064
LIN = 16 * VP
W = 7808
NPW = 16
NBUF = 2


def _detile_body(vu_hbm, vi_hbm, lu_hbm, li_hbm,
                 buf0, buf1, s0, s1, so0, so1):
    wid = lax.axis_index("s") * NC + lax.axis_index("c")
    tbl = wid // 16
    rem = wid % 16
    tr = rem // 8
    o = rem % 8
    bufs = (buf0, buf1)
    sin = (s0, s1)
    sout = (so0, so1)

    W1 = 3968

    def run(src, dst):
        def start_in(i, b):
            c0 = (o * NPW + i) * W
            return [
                pltpu.async_copy(src.at[tr, :, pl.ds(c0, W1)],
                                 bufs[b].at[:, pl.ds(0, W1)], sin[b]),
                pltpu.async_copy(src.at[tr, :, pl.ds(c0 + W1, W - W1)],
                                 bufs[b].at[:, pl.ds(W1, W - W1)], sin[b]),
            ]

        in_flight = [None] * NBUF
        out_flight = [[] for _ in range(NBUF)]
        for p in range(NBUF - 1):
            in_flight[p] = start_in(p, p)
        for i in range(NPW):
            b = i % NBUF
            for c in in_flight[b]:
                c.wait()
            nxt = i + NBUF - 1
            if nxt < NPW:
                nb = nxt % NBUF
                for c in out_flight[nb]:
                    c.wait()
                in_flight[nb] = start_in(nxt, nb)
            c0 = (o * NPW + i) * W
            out_flight[b] = [
                pltpu.async_copy(bufs[b].at[fr],
                                 dst.at[pl.ds((tr * 8 + fr) * VP + c0, W)],
                                 sout[b])
                for fr in range(8)
            ]
        for plist in out_flight:
            for c in plist:
                c.wait()

        @pl.when(o == 7)
        def _():
            c0 = 8 * NPW * W
            pltpu.sync_copy(src.at[tr, :, pl.ds(c0, 512)],
                            bufs[0].at[:, pl.ds(0, 512)])
            for fr in range(8):
                pltpu.sync_copy(bufs[0].at[fr, pl.ds(0, 512)],
                                dst.at[pl.ds((tr * 8 + fr) * VP + c0, 512)])

    @pl.when(tbl == 0)
    def _():
        run(vu_hbm, lu_hbm)

    @pl.when(tbl == 1)
    def _():
        run(vi_hbm, li_hbm)


TS = V - 64


def _gather_body(users_hbm, items_hbm, lu_hbm, li_hbm, ub_hbm, ib_hbm,
                 tu_hbm, ti_hbm, out_hbm,
                 idx_u, idx_i, cols_u, cols_i, bias_u, bias_i, out_v,
                 tail_u, tail_i, sem):
    wid = lax.axis_index("s") * NC + lax.axis_index("c")
    base = wid * BPW

    pltpu.sync_copy(users_hbm.at[pl.ds(base, BPW)], idx_u)
    pltpu.sync_copy(items_hbm.at[pl.ds(base, BPW)], idx_i)
    pltpu.sync_copy(tu_hbm, tail_u)
    pltpu.sync_copy(ti_hbm, tail_i)

    copies = [
        pltpu.async_copy(ub_hbm.at[idx_u], bias_u, sem),
        pltpu.async_copy(ib_hbm.at[idx_i], bias_i, sem),
    ]
    for f in range(D):
        copies.append(pltpu.async_copy(
            lu_hbm.at[pl.ds(f * VP, V)].at[idx_u], cols_u.at[f], sem))
        copies.append(pltpu.async_copy(
            li_hbm.at[pl.ds(f * VP, V)].at[idx_i], cols_i.at[f], sem))
    for c in copies:
        c.wait()

    @pl.loop(0, G)
    def _(g):
        r0 = g * D
        iu = idx_u[pl.ds(r0, D)]
        ii = idx_i[pl.ds(r0, D)]
        mu = iu >= TS
        mi = ii >= TS
        tix_u = jnp.maximum(iu - TS, 0)
        tix_i = jnp.maximum(ii - TS, 0)
        acc = bias_u[pl.ds(r0, D)] + bias_i[pl.ds(r0, D)]
        for f in range(D):
            uf = cols_u[f, pl.ds(r0, D)]
            vf = cols_i[f, pl.ds(r0, D)]
            uf = jnp.where(mu, plsc.load_gather(tail_u, [tix_u + f * 64]), uf)
            vf = jnp.where(mi, plsc.load_gather(tail_i, [tix_i + f * 64]), vf)
            acc = acc + uf * vf
        out_v[pl.ds(r0, D)] = acc

    pltpu.sync_copy(out_v, out_hbm.at[pl.ds(base, BPW)])


def kernel(users, items, user_embeddings, item_embeddings, user_biases, item_biases):
    users = users.astype(jnp.int32)
    items = items.astype(jnp.int32)
    vu = user_embeddings.T.reshape(NC, 8, V)
    vi = item_embeddings.T.reshape(NC, 8, V)
    ub = user_biases.reshape(-1)
    ib = item_biases.reshape(-1)
    tu = lax.slice(user_embeddings, (TS, 0), (V, D)).T.reshape(-1)
    ti = lax.slice(item_embeddings, (TS, 0), (V, D)).T.reshape(-1)

    mesh = plsc.VectorSubcoreMesh(core_axis_name="c", subcore_axis_name="s")

    detile = pl.kernel(
        _detile_body,
        out_type=(
            jax.ShapeDtypeStruct((LIN,), jnp.float32),
            jax.ShapeDtypeStruct((LIN,), jnp.float32),
        ),
        mesh=mesh,
        scratch_types=[
            pltpu.VMEM((8, W), jnp.float32),
            pltpu.VMEM((8, W), jnp.float32),
            pltpu.SemaphoreType.DMA,
            pltpu.SemaphoreType.DMA,
            pltpu.SemaphoreType.DMA,
            pltpu.SemaphoreType.DMA,
        ],
        compiler_params=pltpu.CompilerParams(
            needs_layout_passes=False, use_tc_tiling_on_sc=True),
    )
    lu, li = detile(vu, vi)

    gather = pl.kernel(
        _gather_body,
        out_type=jax.ShapeDtypeStruct((B,), jnp.float32),
        mesh=mesh,
        scratch_types=[
            pltpu.VMEM((BPW,), jnp.int32),
            pltpu.VMEM((BPW,), jnp.int32),
            pltpu.VMEM((D, BPW), jnp.float32),
            pltpu.VMEM((D, BPW), jnp.float32),
            pltpu.VMEM((BPW,), jnp.float32),
            pltpu.VMEM((BPW,), jnp.float32),
            pltpu.VMEM((BPW,), jnp.float32),
            pltpu.VMEM((D * 64,), jnp.float32),
            pltpu.VMEM((D * 64,), jnp.float32),
            pltpu.SemaphoreType.DMA,
        ],
        compiler_params=pltpu.CompilerParams(
            needs_layout_passes=False, use_tc_tiling_on_sc=False),
    )
    return gather(users, items, lu, li, ub, ib, tu, ti)


# speedup vs baseline: 1.0067x; 1.0067x over previous
import jax
import jax.numpy as jnp
from jax import lax
from jax.experimental import pallas as pl
from jax.experimental.pallas import tpu as pltpu
from jax.experimental.pallas import tpu_sc as plsc

B = 16384
D = 16
NC = 2
NS = 16
NW = NC * NS
BPW = B // NW
G = BPW // D

V = 1_000_000
VP = 1_000_064
LIN = 16 * VP
W = 7808
NPW = 16
NBUF = 2


def _detile_body(vu_hbm, vi_hbm, lu_hbm, li_hbm,
                 buf0, buf1, s0, s1, so0, so1):
    wid = lax.axis_index("s") * NC + lax.axis_index("c")
    tbl = wid // 16
    rem = wid % 16
    tr = rem // 8
    o = rem % 8
    bufs = (buf0, buf1)
    sin = (s0, s1)
    sout = (so0, so1)

    def run(src, dst):
        def chunk_src(i):
            return src.at[tr, :, pl.ds((o * NPW + i) * W, W)]

        in_flight = [None] * NBUF
        out_flight = [[] for _ in range(NBUF)]
        for p in range(NBUF - 1):
            in_flight[p] = pltpu.async_copy(chunk_src(p), bufs[p], sin[p])
        for i in range(NPW):
            b = i % NBUF
            in_flight[b].wait()
            nxt = i + NBUF - 1
            if nxt < NPW:
                nb = nxt % NBUF
                for c in out_flight[nb]:
                    c.wait()
                in_flight[nb] = pltpu.async_copy(chunk_src(nxt), bufs[nb], sin[nb])
            c0 = (o * NPW + i) * W
            out_flight[b] = [
                pltpu.async_copy(bufs[b].at[fr],
                                 dst.at[pl.ds((tr * 8 + fr) * VP + c0, W)],
                                 sout[b])
                for fr in range(8)
            ]
        for plist in out_flight:
            for c in plist:
                c.wait()

        @pl.when(o == 7)
        def _():
            c0 = 8 * NPW * W
            pltpu.sync_copy(src.at[tr, :, pl.ds(c0, 512)],
                            bufs[0].at[:, pl.ds(0, 512)])
            for fr in range(8):
                pltpu.sync_copy(bufs[0].at[fr, pl.ds(0, 512)],
                                dst.at[pl.ds((tr * 8 + fr) * VP + c0, 512)])

    @pl.when(tbl == 0)
    def _():
        run(vu_hbm, lu_hbm)

    @pl.when(tbl == 1)
    def _():
        run(vi_hbm, li_hbm)


TS = V - 64


def _gather_body(users_hbm, items_hbm, lu_hbm, li_hbm, ub_hbm, ib_hbm,
                 tu_hbm, ti_hbm, out_hbm,
                 idx_u, idx_i, cols_u, cols_i, bias_u, bias_i, out_v,
                 tail_u, tail_i, sem):
    wid = lax.axis_index("s") * NC + lax.axis_index("c")
    base = wid * BPW

    pltpu.sync_copy(users_hbm.at[pl.ds(base, BPW)], idx_u)
    pltpu.sync_copy(items_hbm.at[pl.ds(base, BPW)], idx_i)
    pltpu.sync_copy(tu_hbm, tail_u)
    pltpu.sync_copy(ti_hbm, tail_i)

    copies = [
        pltpu.async_copy(ub_hbm.at[idx_u], bias_u, sem),
        pltpu.async_copy(ib_hbm.at[idx_i], bias_i, sem),
    ]
    for f in range(D):
        copies.append(pltpu.async_copy(
            lu_hbm.at[pl.ds(f * VP, V)].at[idx_u], cols_u.at[f], sem))
        copies.append(pltpu.async_copy(
            li_hbm.at[pl.ds(f * VP, V)].at[idx_i], cols_i.at[f], sem))
    for c in copies:
        c.wait()

    @pl.loop(0, G)
    def _(g):
        r0 = g * D
        iu = idx_u[pl.ds(r0, D)]
        ii = idx_i[pl.ds(r0, D)]
        mu = iu >= TS
        mi = ii >= TS
        tix_u = jnp.maximum(iu - TS, 0)
        tix_i = jnp.maximum(ii - TS, 0)
        acc = bias_u[pl.ds(r0, D)] + bias_i[pl.ds(r0, D)]
        for f in range(D):
            uf = cols_u[f, pl.ds(r0, D)]
            vf = cols_i[f, pl.ds(r0, D)]
            uf = jnp.where(mu, plsc.load_gather(tail_u, [tix_u + f * 64]), uf)
            vf = jnp.where(mi, plsc.load_gather(tail_i, [tix_i + f * 64]), vf)
            acc = acc + uf * vf
        out_v[pl.ds(r0, D)] = acc

    pltpu.sync_copy(out_v, out_hbm.at[pl.ds(base, BPW)])


def kernel(users, items, user_embeddings, item_embeddings, user_biases, item_biases):
    users = users.astype(jnp.int32)
    items = items.astype(jnp.int32)
    vu = user_embeddings.T.reshape(NC, 8, V)
    vi = item_embeddings.T.reshape(NC, 8, V)
    ub = user_biases.reshape(-1)
    ib = item_biases.reshape(-1)
    tu = lax.slice(user_embeddings, (TS, 0), (V, D)).T.reshape(-1)
    ti = lax.slice(item_embeddings, (TS, 0), (V, D)).T.reshape(-1)

    mesh = plsc.VectorSubcoreMesh(core_axis_name="c", subcore_axis_name="s")

    detile = pl.kernel(
        _detile_body,
        out_type=(
            jax.ShapeDtypeStruct((LIN,), jnp.float32),
            jax.ShapeDtypeStruct((LIN,), jnp.float32),
        ),
        mesh=mesh,
        scratch_types=[
            pltpu.VMEM((8, W), jnp.float32),
            pltpu.VMEM((8, W), jnp.float32),
            pltpu.SemaphoreType.DMA,
            pltpu.SemaphoreType.DMA,
            pltpu.SemaphoreType.DMA,
            pltpu.SemaphoreType.DMA,
        ],
        compiler_params=pltpu.CompilerParams(
            needs_layout_passes=False, use_tc_tiling_on_sc=True),
    )
    lu, li = detile(vu, vi)

    gather = pl.kernel(
        _gather_body,
        out_type=jax.ShapeDtypeStruct((B,), jnp.float32),
        mesh=mesh,
        scratch_types=[
            pltpu.VMEM((BPW,), jnp.int32),
            pltpu.VMEM((BPW,), jnp.int32),
            pltpu.VMEM((D, BPW), jnp.float32),
            pltpu.VMEM((D, BPW), jnp.float32),
            pltpu.VMEM((BPW,), jnp.float32),
            pltpu.VMEM((BPW,), jnp.float32),
            pltpu.VMEM((BPW,), jnp.float32),
            pltpu.VMEM((D * 64,), jnp.float32),
            pltpu.VMEM((D * 64,), jnp.float32),
            pltpu.SemaphoreType.DMA,
        ],
        compiler_params=pltpu.CompilerParams(
            needs_layout_passes=False, use_tc_tiling_on_sc=False),
    )
    return gather(users, items, lu, li, ub, ib, tu, ti)
